# baseline (speedup 1.0000x reference)
import jax
import jax.numpy as jnp
from jax import lax
from jax.experimental import pallas as pl
from jax.experimental.pallas import tpu as pltpu

_INTEGRAL = 1.0
_BB = 2048


def _cnf_kernel(t_ref, z_ref, w1_ref, b1_ref, w2_ref, b2_ref, f_ref, dl_ref):
    t = t_ref[0]
    z = z_ref[...]
    w1 = w1_ref[...]
    w1z = w1[1:, :]
    w2 = w2_ref[...]

    a = -_INTEGRAL * t
    tm = jnp.full((1, 1), t, dtype=jnp.float32)
    b = a * lax.rsqrt(1.0 - jnp.exp(-(_INTEGRAL * tm * tm)))

    pre = jnp.dot(z, w1z, preferred_element_type=jnp.float32)
    pre = pre + (t * w1[0:1, :] + b1_ref[...])
    h = jnp.tanh(pre)

    g = jnp.dot(w2, w1z, preferred_element_type=jnp.float32)
    hh = g.shape[0]
    rows = lax.broadcasted_iota(jnp.int32, (hh, hh), 0)
    cols = lax.broadcasted_iota(jnp.int32, (hh, hh), 1)
    c = jnp.sum(jnp.where(rows == cols, g, 0.0), axis=0, keepdims=True)
    c0 = jnp.sum(c, axis=1, keepdims=True)
    bc = b * c

    hf = h.shape[1] // 2
    h1, h2 = h[:, :hf], h[:, hf:]
    tr1 = jnp.sum((h1 * h1) * bc[:, :hf], axis=1, keepdims=True)
    mlp1 = jnp.dot(h1, w2[:hf, :], preferred_element_type=jnp.float32)
    tr2 = jnp.sum((h2 * h2) * bc[:, hf:], axis=1, keepdims=True)
    mlp2 = jnp.dot(h2, w2[hf:, :], preferred_element_type=jnp.float32)
    f_ref[...] = a * z - b * (mlp1 + mlp2 + b2_ref[...])
    dl_ref[...] = (b * c0 - a * jnp.float32(z.shape[1])) - (tr1 + tr2)


def kernel(t, z, W1, b1, W2, b2):
    B, D = z.shape
    H = W2.shape[0]

    grid = (B // _BB,)
    f, dl = pl.pallas_call(
        _cnf_kernel,
        grid=grid,
        in_specs=[
            pl.BlockSpec(memory_space=pltpu.SMEM),
            pl.BlockSpec((_BB, D), lambda i: (i, 0)),
            pl.BlockSpec((D + 1, H), lambda i: (0, 0)),
            pl.BlockSpec((1, H), lambda i: (0, 0)),
            pl.BlockSpec((H, D), lambda i: (0, 0)),
            pl.BlockSpec((1, D), lambda i: (0, 0)),
        ],
        out_specs=[
            pl.BlockSpec((_BB, D), lambda i: (i, 0)),
            pl.BlockSpec((_BB, 1), lambda i: (i, 0)),
        ],
        out_shape=[
            jax.ShapeDtypeStruct((B, D), jnp.float32),
            jax.ShapeDtypeStruct((B, 1), jnp.float32),
        ],
        compiler_params=pltpu.CompilerParams(
            dimension_semantics=("parallel",),
        ),
        name="cnf_trace_fused",
    )(t, z, W1, b1.reshape(1, H), W2, b2.reshape(1, D))
    return f, dl


# speedup vs baseline: 1.6194x; 1.6194x over previous
import jax
import jax.numpy as jnp
from jax import lax
from jax.experimental import pallas as pl
from jax.experimental.pallas import tpu as pltpu

_INTEGRAL = 1.0
_BB = 2048


def _cnf_kernel(t_ref, zt_ref, w1t_ref, b1c_ref, w2t_ref, b2c_ref,
                ft_ref, dlt_ref):
    t = t_ref[0]
    zt = zt_ref[...]
    w1t = w1t_ref[...]
    w1zt = w1t[:, 1:]
    w2t = w2t_ref[...]

    a = -_INTEGRAL * t
    tm = jnp.full((1, 1), t, dtype=jnp.float32)
    b = a * lax.rsqrt(1.0 - jnp.exp(-(_INTEGRAL * tm * tm)))

    pre = jnp.dot(w1zt, zt, preferred_element_type=jnp.float32)
    pre = pre + (t * w1t[:, 0:1] + b1c_ref[...])
    h = jnp.tanh(pre)
    mlp = jnp.dot(w2t, h, preferred_element_type=jnp.float32) + b2c_ref[...]
    ft_ref[...] = a * zt - b * mlp

    g = jnp.dot(w1zt, w2t, preferred_element_type=jnp.float32)
    hh = g.shape[0]
    rows = lax.broadcasted_iota(jnp.int32, (hh, hh), 0)
    cols = lax.broadcasted_iota(jnp.int32, (hh, hh), 1)
    c_col = jnp.sum(jnp.where(rows == cols, g, 0.0), axis=1, keepdims=True)
    c0 = jnp.sum(c_col, axis=0, keepdims=True)
    tr_neg = jnp.sum((h * h) * (b * c_col), axis=0, keepdims=True)
    dlt_ref[...] = (b * c0 - a * jnp.float32(zt.shape[0])) - tr_neg


def kernel(t, z, W1, b1, W2, b2):
    B, D = z.shape
    H = W2.shape[0]

    zt = z.T
    w1t = W1.T
    w2t = W2.T
    b1c = b1.reshape(H, 1)
    b2c = b2.reshape(D, 1)

    grid = (B // _BB,)
    ft, dlt = pl.pallas_call(
        _cnf_kernel,
        grid=grid,
        in_specs=[
            pl.BlockSpec(memory_space=pltpu.SMEM),
            pl.BlockSpec((D, _BB), lambda i: (0, i)),
            pl.BlockSpec((H, D + 1), lambda i: (0, 0)),
            pl.BlockSpec((H, 1), lambda i: (0, 0)),
            pl.BlockSpec((D, H), lambda i: (0, 0)),
            pl.BlockSpec((D, 1), lambda i: (0, 0)),
        ],
        out_specs=[
            pl.BlockSpec((D, _BB), lambda i: (0, i)),
            pl.BlockSpec((1, _BB), lambda i: (0, i)),
        ],
        out_shape=[
            jax.ShapeDtypeStruct((D, B), jnp.float32),
            jax.ShapeDtypeStruct((1, B), jnp.float32),
        ],
        compiler_params=pltpu.CompilerParams(
            dimension_semantics=("parallel",),
        ),
        name="cnf_trace_fused_t",
    )(t, zt, w1t, b1c, w2t, b2c)
    return ft.T, dlt.T
